# baseline (speedup 1.0000x reference)
import functools

import jax
import jax.numpy as jnp
from jax import lax
from jax.experimental import pallas as pl
from jax.experimental.pallas import tpu as pltpu
from jax.experimental.pallas import tpu_sc as plsc

N_FIELDS = 26
VOCAB1 = 100001
EMB = 64
BATCH = 4096
NW = 32
CHUNK = BATCH // NW
LANES = 16

_mesh = plsc.VectorSubcoreMesh(core_axis_name="c", subcore_axis_name="s")


@functools.partial(
    pl.kernel,
    mesh=_mesh,
    compiler_params=pltpu.CompilerParams(use_tc_tiling_on_sc=False),
    out_type=jax.ShapeDtypeStruct((N_FIELDS, BATCH, EMB), jnp.float32),
    scratch_types=[
        pltpu.VMEM((CHUNK,), jnp.int32),
        pltpu.VMEM((CHUNK, EMB), jnp.float32),
        pltpu.SemaphoreType.DMA,
    ],
)
def _gather_kernel(xs_hbm, tab_hbm, out_hbm, idx_v, rows_v, sem):
    wid = lax.axis_index("s") * 2 + lax.axis_index("c")
    col0 = wid * CHUNK

    def body(f, carry):
        pltpu.sync_copy(xs_hbm.at[f, pl.ds(col0, CHUNK)], idx_v)
        pltpu.async_copy(tab_hbm.at[f].at[idx_v], rows_v, sem).wait()
        pltpu.sync_copy(rows_v, out_hbm.at[f, pl.ds(col0, CHUNK)])
        return carry

    lax.fori_loop(0, N_FIELDS, body, 0)


def kernel(xs, W):
    return _gather_kernel(xs, W)


# speedup vs baseline: 8.3033x; 8.3033x over previous
import functools

import jax
import jax.numpy as jnp
from jax import lax
from jax.experimental import pallas as pl
from jax.experimental.pallas import tpu as pltpu
from jax.experimental.pallas import tpu_sc as plsc

N_FIELDS = 26
VOCAB1 = 100001
EMB = 64
BATCH = 4096
NW = 32
CHUNK = BATCH // NW
LANES = 16
NSLOT = 4

_mesh = plsc.VectorSubcoreMesh(core_axis_name="c", subcore_axis_name="s")


@functools.partial(
    pl.kernel,
    mesh=_mesh,
    compiler_params=pltpu.CompilerParams(needs_layout_passes=False),
    out_type=jax.ShapeDtypeStruct((N_FIELDS, BATCH, EMB), jnp.float32),
    scratch_types=[
        pltpu.VMEM((N_FIELDS, CHUNK), jnp.int32),
        pltpu.VMEM((NSLOT, CHUNK, EMB), jnp.float32),
        pltpu.SemaphoreType.DMA((NSLOT,)),
        pltpu.SemaphoreType.DMA((NSLOT,)),
    ],
)
def _gather_kernel(xs_hbm, tab_hbm, out_hbm, idx2_v, rows8_v, gsem, wsem):
    wid = lax.axis_index("s") * 2 + lax.axis_index("c")
    col0 = wid * CHUNK
    lane_iota = lax.iota(jnp.int32, LANES)

    pltpu.sync_copy(xs_hbm.at[:, pl.ds(col0, CHUNK)], idx2_v)

    def issue_chunk(f, slot):
        tab_f = tab_hbm.at[f]

        def grp(g, c):
            v16 = idx2_v[f, pl.ds(g * LANES, LANES)]
            for l in range(LANES):
                r = jnp.sum(jnp.where(lane_iota == l, v16, 0))
                pltpu.make_async_copy(
                    tab_f.at[pl.ds(r, 1)],
                    rows8_v.at[slot, pl.ds(g * LANES + l, 1)],
                    gsem.at[slot],
                ).start()
            return c

        lax.fori_loop(0, CHUNK // LANES, grp, 0)

    def drain_and_wb(g, slot):
        pltpu.make_async_copy(
            tab_hbm.at[0].at[pl.ds(0, CHUNK)], rows8_v.at[slot], gsem.at[slot]
        ).wait()
        pltpu.make_async_copy(
            rows8_v.at[slot], out_hbm.at[g, pl.ds(col0, CHUNK)], wsem.at[slot]
        ).start()

    def body(f, carry):
        slot = lax.rem(f, NSLOT)

        @pl.when(f >= NSLOT)
        def _():
            pltpu.make_async_copy(
                rows8_v.at[slot],
                out_hbm.at[f - NSLOT, pl.ds(col0, CHUNK)],
                wsem.at[slot],
            ).wait()

        issue_chunk(f, slot)

        @pl.when(f >= 2)
        def _():
            g = f - 2
            drain_and_wb(g, lax.rem(g, NSLOT))

        return carry

    lax.fori_loop(0, N_FIELDS, body, 0)

    def tail(t, carry):
        g = N_FIELDS - 2 + t
        drain_and_wb(g, lax.rem(g, NSLOT))
        return carry

    lax.fori_loop(0, 2, tail, 0)

    def tail_wb(t, carry):
        g = N_FIELDS - NSLOT + t
        slot = lax.rem(g, NSLOT)
        pltpu.make_async_copy(
            rows8_v.at[slot], out_hbm.at[g, pl.ds(col0, CHUNK)], wsem.at[slot]
        ).wait()
        return carry

    lax.fori_loop(0, NSLOT, tail_wb, 0)


def kernel(xs, W):
    return _gather_kernel(xs, W)
